# Initial kernel scaffold; baseline (speedup 1.0000x reference)
import jax
import jax.numpy as jnp
from jax.experimental import pallas as pl


def kernel(x, table):
    raise NotImplementedError("write your pallas kernel here")


# speedup vs baseline: 1.3533x; 1.3533x over previous
import functools

import jax
import jax.numpy as jnp
from jax import lax
from jax.experimental import pallas as pl
from jax.experimental.pallas import tpu as pltpu
from jax.experimental.pallas import tpu_sc as plsc

NC, NS, L = 2, 16, 16
NW = NC * NS
BATCH, SEQ = 4096, 200
N_ELEMS = BATCH * SEQ
PER_W = N_ELEMS // NW
CHUNK = 512
N_CHUNKS = PER_W // CHUNK
GROUPS = CHUNK // L
EMBED = 64
NROWS = 34


def _sc_body(x_hbm, table_hbm, out_hbm, table_v, x_v, out_v):
    wid = lax.axis_index("s") * NC + lax.axis_index("c")
    base_elem = wid * PER_W
    pltpu.sync_copy(table_hbm, table_v)
    lane = lax.broadcasted_iota(jnp.int32, (L,), 0)

    def chunk_body(ci, carry):
        e0 = base_elem + ci * CHUNK
        pltpu.sync_copy(x_hbm.at[pl.ds(e0, CHUNK)], x_v)

        def group_body(gi, carry2):
            xv = x_v[pl.ds(gi * L, L)]
            nan = xv != xv
            t = jnp.clip((xv + 4.0) * 4.0, -1.0, 33.0)
            t = jnp.where(nan, 0.0, t)
            g = jnp.clip(t.astype(jnp.int32), 0, 32)
            bg = g.astype(jnp.float32) * 0.25 - 4.0
            inc = jnp.where(xv >= bg + 0.25, 1, 0)
            dec = jnp.where(xv < bg, 1, 0)
            idx = jnp.clip(g + inc - dec, 0, 32) + 1
            rows = jnp.where(nan, 0, idx)
            dst = gi * L + lane
            for c in range(EMBED):
                cols = jnp.full((L,), c, jnp.int32)
                vals = plsc.load_gather(table_v, [rows, cols])
                plsc.store_scatter(out_v, [dst, cols], vals)
            return carry2

        lax.fori_loop(0, GROUPS, group_body, 0)
        pltpu.sync_copy(out_v, out_hbm.at[pl.ds(e0, CHUNK)])
        return carry

    lax.fori_loop(0, N_CHUNKS, chunk_body, 0)


_sc_embed = functools.partial(
    pl.kernel,
    out_type=jax.ShapeDtypeStruct((N_ELEMS, EMBED), jnp.float32),
    mesh=plsc.VectorSubcoreMesh(core_axis_name="c", subcore_axis_name="s"),
    compiler_params=pltpu.CompilerParams(needs_layout_passes=False),
    scratch_types=[
        pltpu.VMEM((NROWS, EMBED), jnp.float32),
        pltpu.VMEM((CHUNK,), jnp.float32),
        pltpu.VMEM((CHUNK, EMBED), jnp.float32),
    ],
)(_sc_body)


def kernel(x, table):
    out = _sc_embed(x.reshape(N_ELEMS), table)
    return out.reshape(BATCH, SEQ, EMBED)


# speedup vs baseline: 6.9735x; 5.1529x over previous
import functools

import jax
import jax.numpy as jnp
from jax import lax
from jax.experimental import pallas as pl
from jax.experimental.pallas import tpu as pltpu
from jax.experimental.pallas import tpu_sc as plsc

NC, NS, L = 2, 16, 16
NW = NC * NS
BATCH, SEQ = 4096, 200
N_ELEMS = BATCH * SEQ
PER_W = N_ELEMS // NW
CHUNK = 512
N_CHUNKS = PER_W // CHUNK
GROUPS = CHUNK // L
IDX_ROWS = CHUNK // 128
EMBED = 64
NROWS = 34


def _bin_rows(xv):
    nan = xv != xv
    t = jnp.clip((xv + 4.0) * 4.0, -1.0, 33.0)
    t = jnp.where(nan, 0.0, t)
    g = jnp.clip(t.astype(jnp.int32), 0, 32)
    bg = g.astype(jnp.float32) * 0.25 - 4.0
    inc = jnp.where(xv >= bg + 0.25, 1, 0)
    dec = jnp.where(xv < bg, 1, 0)
    idx = jnp.clip(g + inc - dec, 0, 32) + 1
    return jnp.where(nan, 0, idx)


def _sc_body(x_hbm, table_hbm, out_hbm, table_sh, x_v, idx_v, out_v, sem):
    cid = lax.axis_index("c")
    sid = lax.axis_index("s")
    wid = sid * NC + cid
    base_elem = wid * PER_W

    @pl.when(sid == 0)
    def _copy_table():
        pltpu.sync_copy(table_hbm, table_sh)

    plsc.subcore_barrier()

    def chunk_body(ci, carry):
        e0 = base_elem + ci * CHUNK
        pltpu.sync_copy(x_hbm.at[pl.ds(e0, CHUNK)], x_v)
        for gi in range(GROUPS):
            xv = x_v[pl.ds(gi * L, L)]
            idx_v[gi // 8, pl.ds((gi % 8) * L, L)] = _bin_rows(xv)
        descs = [
            pltpu.async_copy(
                table_sh.at[idx_v.at[j]], out_v.at[pl.ds(j * 128, 128)], sem
            )
            for j in range(IDX_ROWS)
        ]
        for d in descs:
            d.wait()
        pltpu.sync_copy(out_v, out_hbm.at[pl.ds(e0, CHUNK)])
        return carry

    lax.fori_loop(0, N_CHUNKS, chunk_body, 0)


_sc_embed = functools.partial(
    pl.kernel,
    out_type=jax.ShapeDtypeStruct((N_ELEMS, EMBED), jnp.float32),
    mesh=plsc.VectorSubcoreMesh(core_axis_name="c", subcore_axis_name="s"),
    compiler_params=pltpu.CompilerParams(needs_layout_passes=False),
    scratch_types=[
        pltpu.VMEM_SHARED((NROWS, EMBED), jnp.float32),
        pltpu.VMEM((CHUNK,), jnp.float32),
        pltpu.VMEM((IDX_ROWS, 128), jnp.int32),
        pltpu.VMEM((CHUNK, EMBED), jnp.float32),
        pltpu.SemaphoreType.DMA,
    ],
)(_sc_body)


def kernel(x, table):
    out = _sc_embed(x.reshape(N_ELEMS), table)
    return out.reshape(BATCH, SEQ, EMBED)


# speedup vs baseline: 6.9762x; 1.0004x over previous
import functools

import jax
import jax.numpy as jnp
from jax import lax
from jax.experimental import pallas as pl
from jax.experimental.pallas import tpu as pltpu
from jax.experimental.pallas import tpu_sc as plsc

NC, NS, L = 2, 16, 16
NW = NC * NS
BATCH, SEQ = 4096, 200
N_ELEMS = BATCH * SEQ
PER_W = N_ELEMS // NW
CHUNK = 512
N_CHUNKS = PER_W // CHUNK
GROUPS = CHUNK // L
IDX_ROWS = CHUNK // 128
EMBED = 64
NROWS = 34


def _bin_rows(xv):
    nan = xv != xv
    t = jnp.clip((xv + 4.0) * 4.0, -1.0, 33.0)
    t = jnp.where(nan, 0.0, t)
    g = jnp.clip(t.astype(jnp.int32), 0, 32)
    bg = g.astype(jnp.float32) * 0.25 - 4.0
    inc = jnp.where(xv >= bg + 0.25, 1, 0)
    dec = jnp.where(xv < bg, 1, 0)
    idx = jnp.clip(g + inc - dec, 0, 32) + 1
    return jnp.where(nan, 0, idx)


def _sc_body(x_hbm, table_hbm, out_hbm, table_sh, x_v, idx_v, out_v, sem):
    cid = lax.axis_index("c")
    sid = lax.axis_index("s")
    wid = sid * NC + cid
    base_elem = wid * PER_W

    @pl.when(sid == 0)
    def _copy_table():
        pltpu.sync_copy(table_hbm, table_sh)

    plsc.subcore_barrier()

    def chunk_body(ci, carry):
        e0 = base_elem + ci * CHUNK
        pltpu.sync_copy(x_hbm.at[pl.ds(e0, CHUNK)], x_v)
        for gi in range(GROUPS):
            xv = x_v[pl.ds(gi * L, L)]
            idx_v[gi // 8, pl.ds((gi % 8) * L, L)] = _bin_rows(xv)
        descs = [
            pltpu.async_copy(
                table_sh.at[idx_v.at[j]], out_v.at[pl.ds(j * 128, 128)], sem
            )
            for j in range(IDX_ROWS)
        ]
        for d in descs:
            d.wait()
        pltpu.sync_copy(out_v, out_hbm.at[pl.ds(e0, CHUNK)])
        return carry

    lax.fori_loop(0, N_CHUNKS, chunk_body, 0)


_sc_embed = functools.partial(
    pl.kernel,
    out_type=jax.ShapeDtypeStruct((N_ELEMS, EMBED), jnp.float32),
    mesh=plsc.VectorSubcoreMesh(core_axis_name="c", subcore_axis_name="s"),
    compiler_params=pltpu.CompilerParams(needs_layout_passes=False, use_tc_tiling_on_sc=True),
    scratch_types=[
        pltpu.VMEM_SHARED((NROWS, EMBED), jnp.float32),
        pltpu.VMEM((CHUNK,), jnp.float32),
        pltpu.VMEM((IDX_ROWS, 128), jnp.int32),
        pltpu.VMEM((CHUNK, EMBED), jnp.float32),
        pltpu.SemaphoreType.DMA,
    ],
)(_sc_body)


def kernel(x, table):
    out = _sc_embed(x.reshape(N_ELEMS), table)
    return out.reshape(BATCH, SEQ, EMBED)


# speedup vs baseline: 9.0822x; 1.3019x over previous
import functools

import jax
import jax.numpy as jnp
from jax import lax
from jax.experimental import pallas as pl
from jax.experimental.pallas import tpu as pltpu
from jax.experimental.pallas import tpu_sc as plsc

NC, NS, L = 2, 16, 16
NW = NC * NS
BATCH, SEQ = 4096, 200
N_ELEMS = BATCH * SEQ
PER_W = N_ELEMS // NW
CHUNK = 256
N_STEPS = PER_W // (2 * CHUNK)
GROUPS = CHUNK // L
IDX_ROWS = CHUNK // 128
EMBED = 64
NROWS = 34


def _bin_rows(xv):
    nan = xv != xv
    t = jnp.clip((xv + 4.0) * 4.0, -1.0, 33.0)
    t = jnp.where(nan, 0.0, t)
    g = jnp.clip(t.astype(jnp.int32), 0, 32)
    bg = g.astype(jnp.float32) * 0.25 - 4.0
    inc = jnp.where(xv >= bg + 0.25, 1, 0)
    dec = jnp.where(xv < bg, 1, 0)
    idx = jnp.clip(g + inc - dec, 0, 32) + 1
    return jnp.where(nan, 0, idx)


def _sc_body(x_hbm, table_hbm, out_hbm, table_sh, x_v, idx_v, out_v, gsem0, gsem1, osem0, osem1):
    gsem = (gsem0, gsem1)
    osem = (osem0, osem1)
    cid = lax.axis_index("c")
    sid = lax.axis_index("s")
    wid = sid * NC + cid
    base_elem = wid * PER_W

    @pl.when(sid == 0)
    def _copy_table():
        pltpu.sync_copy(table_hbm, table_sh)

    pltpu.sync_copy(x_hbm.at[pl.ds(base_elem, PER_W)], x_v)
    plsc.subcore_barrier()

    def step_body(si, carry):
        for b in range(2):
            ci = si * 2 + b
            e0 = base_elem + ci * CHUNK

            @pl.when(si > 0)
            def _reclaim():
                pltpu.make_async_copy(
                    out_v.at[b], out_hbm.at[pl.ds(0, CHUNK)], osem[b]
                ).wait()

            for gi in range(GROUPS):
                xv = x_v[pl.ds(ci * CHUNK + gi * L, L)]
                idx_v[b, gi // 8, pl.ds((gi % 8) * L, L)] = _bin_rows(xv)
            descs = [
                pltpu.async_copy(
                    table_sh.at[idx_v.at[b, j]],
                    out_v.at[b, pl.ds(j * 128, 128)],
                    gsem[b],
                )
                for j in range(IDX_ROWS)
            ]
            for d in descs:
                d.wait()
            pltpu.async_copy(out_v.at[b], out_hbm.at[pl.ds(e0, CHUNK)], osem[b])
        return carry

    lax.fori_loop(0, N_STEPS, step_body, 0)
    for b in range(2):
        pltpu.make_async_copy(
            out_v.at[b], out_hbm.at[pl.ds(0, CHUNK)], osem[b]
        ).wait()


_sc_embed = functools.partial(
    pl.kernel,
    out_type=jax.ShapeDtypeStruct((N_ELEMS, EMBED), jnp.float32),
    mesh=plsc.VectorSubcoreMesh(core_axis_name="c", subcore_axis_name="s"),
    compiler_params=pltpu.CompilerParams(needs_layout_passes=False),
    scratch_types=[
        pltpu.VMEM_SHARED((NROWS, EMBED), jnp.float32),
        pltpu.VMEM((PER_W,), jnp.float32),
        pltpu.VMEM((2, IDX_ROWS, 128), jnp.int32),
        pltpu.VMEM((2, CHUNK, EMBED), jnp.float32),
        pltpu.SemaphoreType.DMA,
        pltpu.SemaphoreType.DMA,
        pltpu.SemaphoreType.DMA,
        pltpu.SemaphoreType.DMA,
    ],
)(_sc_body)


def kernel(x, table):
    out = _sc_embed(x.reshape(N_ELEMS), table)
    return out.reshape(BATCH, SEQ, EMBED)
